# Initial kernel scaffold; baseline (speedup 1.0000x reference)
import jax
import jax.numpy as jnp
from jax.experimental import pallas as pl


def kernel(x, edge_index, edge_weight, W0, b0, W1, b1, W2, b2, g0, be0, g1, be1, g2, be2):
    raise NotImplementedError("write your pallas kernel here")


# speedup vs baseline: 4.1859x; 4.1859x over previous
import dataclasses
import functools

import jax
import jax.numpy as jnp
from jax import lax
from jax.experimental import pallas as pl
from jax.experimental.pallas import tpu as pltpu
from jax.experimental.pallas import tpu_sc as plsc

NS = 16
LANES = 16
D = 128


def _sc_params():
    cp = pltpu.CompilerParams()
    if "needs_layout_passes" in pltpu.CompilerParams.__dataclass_fields__:
        cp = dataclasses.replace(cp, needs_layout_passes=False)
    return cp


def _make_seg_kernel(n_pad, e_total, e_base, chunk):
    e_per_tile = e_total // NS
    nchunks = e_per_tile // chunk
    stripe = n_pad // NS
    assert stripe % 8 == 0
    zpieces = []
    off = 0
    while off < stripe:
        ln = min(chunk, stripe - off)
        zpieces.append((off, ln))
        off += ln
    mesh = plsc.VectorSubcoreMesh(core_axis_name="c", subcore_axis_name="s",
                                  num_cores=1)

    @functools.partial(
        pl.kernel, mesh=mesh,
        out_type=jax.ShapeDtypeStruct((n_pad, D), jnp.float32),
        scratch_types=[
            pltpu.VMEM((chunk,), jnp.int32),
            pltpu.VMEM((chunk,), jnp.int32),
            pltpu.VMEM((chunk * LANES,), jnp.float32),
            pltpu.VMEM((chunk, D), jnp.float32),
            pltpu.VMEM_SHARED((n_pad, D), jnp.float32),
        ])
    def seg(t_hbm, src_hbm, dst_hbm, ewb_hbm, out_hbm,
            src_v, dst_v, ewb_v, rows_v, acc_sh):
        sid = lax.axis_index("s")
        zero = jnp.zeros((LANES,), jnp.float32)

        @pl.loop(0, chunk)
        def _zero_rows(i):
            for r in range(D // LANES):
                rows_v.at[i, pl.ds(r * LANES, LANES)][...] = zero

        for zoff, zlen in zpieces:
            off = pl.multiple_of(sid * stripe + zoff, 8)
            pltpu.sync_copy(rows_v.at[pl.ds(0, zlen)],
                            acc_sh.at[pl.ds(off, zlen)])
        plsc.subcore_barrier()

        base = e_base + sid * e_per_tile

        @pl.loop(0, nchunks)
        def _chunk_loop(k):
            off = pl.multiple_of(base + k * chunk, 8)
            pltpu.sync_copy(src_hbm.at[pl.ds(off, chunk)], src_v)
            pltpu.sync_copy(dst_hbm.at[pl.ds(off, chunk)], dst_v)
            pltpu.sync_copy(
                ewb_hbm.at[pl.ds(pl.multiple_of(off * LANES, 8),
                                 chunk * LANES)], ewb_v)
            pltpu.sync_copy(t_hbm.at[src_v], rows_v)

            @pl.loop(0, chunk // 8)
            def _scale(g):
                for s in range(8):
                    e = g * 8 + s
                    wv = ewb_v.at[pl.ds(e * LANES, LANES)][...]
                    for r in range(D // LANES):
                        sl = pl.ds(r * LANES, LANES)
                        rows_v.at[e, sl][...] = rows_v.at[e, sl][...] * wv

            pltpu.sync_copy(rows_v, acc_sh.at[dst_v], add=True)

        plsc.subcore_barrier()
        roff = pl.multiple_of(sid * stripe, 8)
        pltpu.sync_copy(acc_sh.at[pl.ds(roff, stripe)],
                        out_hbm.at[pl.ds(roff, stripe)])

    return seg


def _tc_call(body, out_shapes, *args):
    return pl.pallas_call(body, out_shape=out_shapes)(*args)


_BLK = 1000


def _row_spec(d):
    return pl.BlockSpec((_BLK, d), lambda i: (i, 0))


def _full_spec(shape):
    return pl.BlockSpec(shape, lambda i: tuple(0 for _ in shape))


def _ewb_body(ew_ref, out_ref):
    w = ew_ref[...]
    out_ref[...] = jnp.concatenate(
        [jnp.broadcast_to(w[:, j:j + 1], (w.shape[0], LANES))
         for j in range(8)], axis=1)


def _matmul_body(x_ref, w_ref, out_ref):
    out_ref[...] = jnp.dot(x_ref[...], w_ref[...],
                           preferred_element_type=jnp.float32,
                           precision=lax.Precision.HIGHEST)


def _degfin_body(n, dp0_ref, dp1_ref, out_ref):
    deg = dp0_ref[...][:n, 0:1] + dp1_ref[...][:n, 0:1] + 1.0
    out_ref[...] = lax.rsqrt(deg)


def _dinv_body(dcol_ref, h0_ref, dinv_ref, t0_ref):
    dv = jnp.broadcast_to(dcol_ref[...], h0_ref.shape)
    dinv_ref[...] = dv
    t0_ref[...] = dv * h0_ref[...]


def _pre_body(relu, p0_ref, p1_ref, h_ref, dinv_ref, x0_ref, b_ref, out_ref):
    dv = dinv_ref[...]
    agg = (dv * (p0_ref[...] + p1_ref[...]) + dv * dv * h_ref[...]
           + b_ref[...])
    if relu:
        agg = jnp.maximum(agg, 0.0)
    out_ref[...] = agg + x0_ref[...]


def _stats_body(h_ref, out_ref):
    h = h_ref[...]
    m = jnp.mean(h, axis=0, keepdims=True)
    v = jnp.mean((h - m) ** 2, axis=0, keepdims=True)
    out_ref[...] = jnp.concatenate([m, v], axis=0)


def _bnmm_body(bn, has_next, *refs):
    if has_next:
        if bn:
            (h_ref, st_ref, g_ref, be_ref, wn_ref, dinv_ref,
             hn_ref, tn_ref) = refs
        else:
            h_ref, wn_ref, dinv_ref, hn_ref, tn_ref = refs
    else:
        h_ref, st_ref, g_ref, be_ref, out_ref = refs
    h = h_ref[...]
    if bn:
        st = st_ref[...]
        m, v = st[0:1, :], st[1:2, :]
        h = (h - m) / jnp.sqrt(v + 1e-5) * g_ref[...] + be_ref[...]
    if has_next:
        hn = jnp.dot(h, wn_ref[...], preferred_element_type=jnp.float32,
                     precision=lax.Precision.HIGHEST)
        hn_ref[...] = hn
        tn_ref[...] = dinv_ref[...] * hn
    else:
        out_ref[...] = h


def kernel(x, edge_index, edge_weight, W0, b0, W1, b1, W2, b2,
           g0, be0, g1, be1, g2, be2):
    n, d = x.shape
    e = edge_weight.shape[0]
    n_pad = ((n + NS * 8 - 1) // (NS * 8)) * (NS * 8)

    src = edge_index[0]
    dst = edge_index[1]

    f32 = jnp.float32
    sds = jax.ShapeDtypeStruct

    ewb = pl.pallas_call(
        _ewb_body,
        grid=(e // 8 // _BLK,),
        in_specs=[pl.BlockSpec((_BLK, 8), lambda i: (i, 0))],
        out_specs=_row_spec(D),
        out_shape=sds((e // 8, D), f32),
    )(edge_weight.reshape(e // 8, 8)).reshape(e * LANES)

    h0 = _tc_call(_matmul_body, sds((n, d), f32), x, W0)

    seg_half = [_make_seg_kernel(n_pad, e // 2, h * (e // 2), 200)
                for h in range(2)]

    ones_t = jnp.ones((n, d), f32)
    dp0 = seg_half[0](ones_t, src, dst, ewb)
    ones_t2, dp0 = lax.optimization_barrier((ones_t, dp0))
    dp1 = seg_half[1](ones_t2, src, dst, ewb)
    dinv_col = _tc_call(functools.partial(_degfin_body, n),
                        sds((n, 1), f32), dp0, dp1)

    dinv_b, t = _tc_call(_dinv_body, [sds((n, d), f32), sds((n, d), f32)],
                         dinv_col, h0)

    b_ = [b0.reshape(1, d), b1.reshape(1, d), b2.reshape(1, d)]
    g_ = [g0.reshape(1, d), g1.reshape(1, d), g2.reshape(1, d)]
    be_ = [be0.reshape(1, d), be1.reshape(1, d), be2.reshape(1, d)]
    wn_ = [W1, W2, None]

    grid_rows = (n // _BLK,)
    h = h0
    out = None
    for i in range(3):
        p0 = seg_half[0](t, src, dst, ewb)
        t_dep, p0 = lax.optimization_barrier((t, p0))
        p1 = seg_half[1](t_dep, src, dst, ewb)
        relu = i < 2
        bn = i != 1
        hpre = pl.pallas_call(
            functools.partial(_pre_body, relu),
            grid=grid_rows,
            in_specs=[_row_spec(d)] * 5 + [_full_spec((1, d))],
            out_specs=_row_spec(d),
            out_shape=sds((n, d), f32),
        )(p0, p1, h, dinv_b, x, b_[i])
        if bn:
            stats = _tc_call(_stats_body, sds((2, d), f32), hpre)
        if i < 2:
            if bn:
                ins = (hpre, stats, g_[i], be_[i], wn_[i], dinv_b)
                in_specs = [_row_spec(d), _full_spec((2, d)),
                            _full_spec((1, d)), _full_spec((1, d)),
                            _full_spec((d, d)), _row_spec(d)]
            else:
                ins = (hpre, wn_[i], dinv_b)
                in_specs = [_row_spec(d), _full_spec((d, d)), _row_spec(d)]
            h, t = pl.pallas_call(
                functools.partial(_bnmm_body, bn, True),
                grid=grid_rows,
                in_specs=in_specs,
                out_specs=[_row_spec(d)] * 2,
                out_shape=[sds((n, d), f32), sds((n, d), f32)],
            )(*ins)
        else:
            out = pl.pallas_call(
                functools.partial(_bnmm_body, bn, False),
                grid=grid_rows,
                in_specs=[_row_spec(d), _full_spec((2, d)),
                          _full_spec((1, d)), _full_spec((1, d))],
                out_specs=_row_spec(d),
                out_shape=sds((n, d), f32),
            )(hpre, stats, g_[i], be_[i])
    return out


# speedup vs baseline: 4.1883x; 1.0006x over previous
import dataclasses
import functools

import jax
import jax.numpy as jnp
from jax import lax
from jax.experimental import pallas as pl
from jax.experimental.pallas import tpu as pltpu
from jax.experimental.pallas import tpu_sc as plsc

NS = 16
LANES = 16
D = 128


def _sc_params():
    cp = pltpu.CompilerParams()
    if "needs_layout_passes" in pltpu.CompilerParams.__dataclass_fields__:
        cp = dataclasses.replace(cp, needs_layout_passes=False)
    return cp


def _make_seg_kernel(n_pad, e_total, e_base, chunk):
    e_per_tile = e_total // NS
    nchunks = e_per_tile // chunk
    stripe = n_pad // NS
    assert stripe % 8 == 0
    zpieces = []
    off = 0
    while off < stripe:
        ln = min(chunk, stripe - off)
        zpieces.append((off, ln))
        off += ln
    mesh = plsc.VectorSubcoreMesh(core_axis_name="c", subcore_axis_name="s",
                                  num_cores=1)

    @functools.partial(
        pl.kernel, mesh=mesh,
        out_type=jax.ShapeDtypeStruct((n_pad, D), jnp.float32),
        scratch_types=[
            pltpu.VMEM((chunk,), jnp.int32),
            pltpu.VMEM((chunk,), jnp.int32),
            pltpu.VMEM((chunk * LANES,), jnp.float32),
            pltpu.VMEM((chunk, D), jnp.float32),
            pltpu.VMEM_SHARED((n_pad, D), jnp.float32),
        ])
    def seg(t_hbm, src_hbm, dst_hbm, ewb_hbm, out_hbm,
            src_v, dst_v, ewb_v, rows_v, acc_sh):
        sid = lax.axis_index("s")
        zero = jnp.zeros((LANES,), jnp.float32)

        @pl.loop(0, chunk)
        def _zero_rows(i):
            for r in range(D // LANES):
                rows_v.at[i, pl.ds(r * LANES, LANES)][...] = zero

        for zoff, zlen in zpieces:
            off = pl.multiple_of(sid * stripe + zoff, 8)
            pltpu.sync_copy(rows_v.at[pl.ds(0, zlen)],
                            acc_sh.at[pl.ds(off, zlen)])
        plsc.subcore_barrier()

        base = e_base + sid * e_per_tile

        @pl.loop(0, nchunks)
        def _chunk_loop(k):
            off = pl.multiple_of(base + k * chunk, 8)
            pltpu.sync_copy(src_hbm.at[pl.ds(off, chunk)], src_v)
            pltpu.sync_copy(dst_hbm.at[pl.ds(off, chunk)], dst_v)
            pltpu.sync_copy(
                ewb_hbm.at[pl.ds(pl.multiple_of(off * LANES, 8),
                                 chunk * LANES)], ewb_v)
            pltpu.sync_copy(t_hbm.at[src_v], rows_v)

            @pl.loop(0, chunk // 8)
            def _scale(g):
                for s in range(8):
                    e = g * 8 + s
                    wv = ewb_v.at[pl.ds(e * LANES, LANES)][...]
                    for r in range(D // LANES):
                        sl = pl.ds(r * LANES, LANES)
                        rows_v.at[e, sl][...] = rows_v.at[e, sl][...] * wv

            pltpu.sync_copy(rows_v, acc_sh.at[dst_v], add=True)

        plsc.subcore_barrier()
        roff = pl.multiple_of(sid * stripe, 8)
        pltpu.sync_copy(acc_sh.at[pl.ds(roff, stripe)],
                        out_hbm.at[pl.ds(roff, stripe)])

    return seg


def _tc_call(body, out_shapes, *args):
    return pl.pallas_call(body, out_shape=out_shapes)(*args)


_BLK = 1000


def _row_spec(d):
    return pl.BlockSpec((_BLK, d), lambda i: (i, 0))


def _full_spec(shape):
    return pl.BlockSpec(shape, lambda i: tuple(0 for _ in shape))


def _ewb_body(ew_ref, out_ref):
    w = ew_ref[...]
    out_ref[...] = jnp.concatenate(
        [jnp.broadcast_to(w[:, j:j + 1], (w.shape[0], LANES))
         for j in range(8)], axis=1)


def _matmul_body(x_ref, w_ref, out_ref):
    out_ref[...] = jnp.dot(x_ref[...], w_ref[...],
                           preferred_element_type=jnp.float32,
                           precision=lax.Precision.HIGHEST)


def _degfin_body(n, dp0_ref, dp1_ref, out_ref):
    deg = dp0_ref[...][:n, 0:1] + dp1_ref[...][:n, 0:1] + 1.0
    out_ref[...] = lax.rsqrt(deg)


def _dinv_body(dcol_ref, h0_ref, dinv_ref, t0_ref):
    dv = jnp.broadcast_to(dcol_ref[...], h0_ref.shape)
    dinv_ref[...] = dv
    t0_ref[...] = dv * h0_ref[...]


def _pre_body(relu, p0_ref, p1_ref, h_ref, dinv_ref, x0_ref, b_ref, out_ref):
    dv = dinv_ref[...]
    agg = (dv * (p0_ref[...] + p1_ref[...]) + dv * dv * h_ref[...]
           + b_ref[...])
    if relu:
        agg = jnp.maximum(agg, 0.0)
    out_ref[...] = agg + x0_ref[...]


def _stats_body(h_ref, out_ref):
    h = h_ref[...]
    m = jnp.mean(h, axis=0, keepdims=True)
    v = jnp.mean((h - m) ** 2, axis=0, keepdims=True)
    out_ref[...] = jnp.concatenate([m, v], axis=0)


def _bnmm_body(bn, has_next, *refs):
    if has_next:
        if bn:
            (h_ref, st_ref, g_ref, be_ref, wn_ref, dinv_ref,
             hn_ref, tn_ref) = refs
        else:
            h_ref, wn_ref, dinv_ref, hn_ref, tn_ref = refs
    else:
        h_ref, st_ref, g_ref, be_ref, out_ref = refs
    h = h_ref[...]
    if bn:
        st = st_ref[...]
        m, v = st[0:1, :], st[1:2, :]
        h = (h - m) / jnp.sqrt(v + 1e-5) * g_ref[...] + be_ref[...]
    if has_next:
        hn = jnp.dot(h, wn_ref[...], preferred_element_type=jnp.float32,
                     precision=lax.Precision.HIGHEST)
        hn_ref[...] = hn
        tn_ref[...] = dinv_ref[...] * hn
    else:
        out_ref[...] = h


def kernel(x, edge_index, edge_weight, W0, b0, W1, b1, W2, b2,
           g0, be0, g1, be1, g2, be2):
    n, d = x.shape
    e = edge_weight.shape[0]
    n_pad = ((n + NS * 8 - 1) // (NS * 8)) * (NS * 8)

    src = edge_index[0]
    dst = edge_index[1]

    f32 = jnp.float32
    sds = jax.ShapeDtypeStruct

    ewb = pl.pallas_call(
        _ewb_body,
        grid=(e // 8 // _BLK,),
        in_specs=[pl.BlockSpec((_BLK, 8), lambda i: (i, 0))],
        out_specs=_row_spec(D),
        out_shape=sds((e // 8, D), f32),
    )(edge_weight.reshape(e // 8, 8)).reshape(e * LANES)

    h0 = _tc_call(_matmul_body, sds((n, d), f32), x, W0)

    seg_half = [_make_seg_kernel(n_pad, e // 2, h * (e // 2), 200)
                for h in range(2)]

    ones_t = jnp.ones((n, d), f32)
    dp0 = seg_half[0](ones_t, src, dst, ewb)
    dp1 = seg_half[1](ones_t, src, dst, ewb)
    dinv_col = _tc_call(functools.partial(_degfin_body, n),
                        sds((n, 1), f32), dp0, dp1)

    dinv_b, t = _tc_call(_dinv_body, [sds((n, d), f32), sds((n, d), f32)],
                         dinv_col, h0)

    b_ = [b0.reshape(1, d), b1.reshape(1, d), b2.reshape(1, d)]
    g_ = [g0.reshape(1, d), g1.reshape(1, d), g2.reshape(1, d)]
    be_ = [be0.reshape(1, d), be1.reshape(1, d), be2.reshape(1, d)]
    wn_ = [W1, W2, None]

    grid_rows = (n // _BLK,)
    h = h0
    out = None
    for i in range(3):
        p0 = seg_half[0](t, src, dst, ewb)
        p1 = seg_half[1](t, src, dst, ewb)
        relu = i < 2
        bn = i != 1
        hpre = pl.pallas_call(
            functools.partial(_pre_body, relu),
            grid=grid_rows,
            in_specs=[_row_spec(d)] * 5 + [_full_spec((1, d))],
            out_specs=_row_spec(d),
            out_shape=sds((n, d), f32),
        )(p0, p1, h, dinv_b, x, b_[i])
        if bn:
            stats = _tc_call(_stats_body, sds((2, d), f32), hpre)
        if i < 2:
            if bn:
                ins = (hpre, stats, g_[i], be_[i], wn_[i], dinv_b)
                in_specs = [_row_spec(d), _full_spec((2, d)),
                            _full_spec((1, d)), _full_spec((1, d)),
                            _full_spec((d, d)), _row_spec(d)]
            else:
                ins = (hpre, wn_[i], dinv_b)
                in_specs = [_row_spec(d), _full_spec((d, d)), _row_spec(d)]
            h, t = pl.pallas_call(
                functools.partial(_bnmm_body, bn, True),
                grid=grid_rows,
                in_specs=in_specs,
                out_specs=[_row_spec(d)] * 2,
                out_shape=[sds((n, d), f32), sds((n, d), f32)],
            )(*ins)
        else:
            out = pl.pallas_call(
                functools.partial(_bnmm_body, bn, False),
                grid=grid_rows,
                in_specs=[_row_spec(d), _full_spec((2, d)),
                          _full_spec((1, d)), _full_spec((1, d))],
                out_specs=_row_spec(d),
                out_shape=sds((n, d), f32),
            )(hpre, stats, g_[i], be_[i])
    return out


# speedup vs baseline: 4.8295x; 1.1531x over previous
import dataclasses
import functools

import jax
import jax.numpy as jnp
from jax import lax
from jax.experimental import pallas as pl
from jax.experimental.pallas import tpu as pltpu
from jax.experimental.pallas import tpu_sc as plsc

NS = 16
LANES = 16
D = 128


def _sc_params():
    cp = pltpu.CompilerParams()
    if "needs_layout_passes" in pltpu.CompilerParams.__dataclass_fields__:
        cp = dataclasses.replace(cp, needs_layout_passes=False)
    return cp


def _make_seg_kernel(n_pad, e_total, e_base, chunk):
    e_per_tile = e_total // NS
    nchunks = e_per_tile // chunk
    stripe = n_pad // NS
    assert stripe % 8 == 0
    zpieces = []
    off = 0
    while off < stripe:
        ln = min(chunk, stripe - off)
        zpieces.append((off, ln))
        off += ln
    mesh = plsc.VectorSubcoreMesh(core_axis_name="c", subcore_axis_name="s",
                                  num_cores=1)

    @functools.partial(
        pl.kernel, mesh=mesh,
        out_type=jax.ShapeDtypeStruct((n_pad, D), jnp.float32),
        scratch_types=[
            pltpu.VMEM((chunk,), jnp.int32),
            pltpu.VMEM((chunk,), jnp.int32),
            pltpu.VMEM((chunk * LANES,), jnp.float32),
            pltpu.VMEM((chunk, D), jnp.float32),
            pltpu.VMEM_SHARED((n_pad, D), jnp.float32),
            pltpu.SemaphoreType.DMA,
        ])
    def seg(t_hbm, src_hbm, dst_hbm, ewb_hbm, out_hbm,
            src_v, dst_v, ewb_v, rows_v, acc_sh, sem_i):
        sid = lax.axis_index("s")
        zero = jnp.zeros((LANES,), jnp.float32)

        @pl.loop(0, chunk)
        def _zero_rows(i):
            for r in range(D // LANES):
                rows_v.at[i, pl.ds(r * LANES, LANES)][...] = zero

        for zoff, zlen in zpieces:
            off = pl.multiple_of(sid * stripe + zoff, 8)
            pltpu.sync_copy(rows_v.at[pl.ds(0, zlen)],
                            acc_sh.at[pl.ds(off, zlen)])
        plsc.subcore_barrier()

        base = e_base + sid * e_per_tile

        @pl.loop(0, nchunks)
        def _chunk_loop(k):
            off = pl.multiple_of(base + k * chunk, 8)
            copies = (
                pltpu.make_async_copy(src_hbm.at[pl.ds(off, chunk)], src_v,
                                      sem_i),
                pltpu.make_async_copy(dst_hbm.at[pl.ds(off, chunk)], dst_v,
                                      sem_i),
                pltpu.make_async_copy(
                    ewb_hbm.at[pl.ds(pl.multiple_of(off * LANES, 8),
                                     chunk * LANES)], ewb_v, sem_i),
            )
            for c in copies:
                c.start()
            for c in copies:
                c.wait()
            pltpu.sync_copy(t_hbm.at[src_v], rows_v)

            @pl.loop(0, chunk // 8)
            def _scale(g):
                for s in range(8):
                    e = g * 8 + s
                    wv = ewb_v.at[pl.ds(e * LANES, LANES)][...]
                    for r in range(D // LANES):
                        sl = pl.ds(r * LANES, LANES)
                        rows_v.at[e, sl][...] = rows_v.at[e, sl][...] * wv

            pltpu.sync_copy(rows_v, acc_sh.at[dst_v], add=True)

        plsc.subcore_barrier()
        roff = pl.multiple_of(sid * stripe, 8)
        pltpu.sync_copy(acc_sh.at[pl.ds(roff, stripe)],
                        out_hbm.at[pl.ds(roff, stripe)])

    return seg


def _tc_call(body, out_shapes, *args):
    return pl.pallas_call(body, out_shape=out_shapes)(*args)


_BLK = 1000


def _row_spec(d):
    return pl.BlockSpec((_BLK, d), lambda i: (i, 0))


def _full_spec(shape):
    return pl.BlockSpec(shape, lambda i: tuple(0 for _ in shape))


def _ewb_body(ew_ref, out_ref):
    w = ew_ref[...]
    out_ref[...] = jnp.concatenate(
        [jnp.broadcast_to(w[:, j:j + 1], (w.shape[0], LANES))
         for j in range(8)], axis=1)


def _matmul_body(x_ref, w_ref, out_ref):
    out_ref[...] = jnp.dot(x_ref[...], w_ref[...],
                           preferred_element_type=jnp.float32,
                           precision=lax.Precision.HIGHEST)


def _degfin_body(n, dp0_ref, dp1_ref, out_ref):
    deg = dp0_ref[...][:n, 0:1] + dp1_ref[...][:n, 0:1] + 1.0
    out_ref[...] = lax.rsqrt(deg)


def _dinv_body(dcol_ref, h0_ref, dinv_ref, t0_ref):
    dv = jnp.broadcast_to(dcol_ref[...], h0_ref.shape)
    dinv_ref[...] = dv
    t0_ref[...] = dv * h0_ref[...]


def _pre_body(relu, p0_ref, p1_ref, h_ref, dinv_ref, x0_ref, b_ref, out_ref):
    dv = dinv_ref[...]
    agg = (dv * (p0_ref[...] + p1_ref[...]) + dv * dv * h_ref[...]
           + b_ref[...])
    if relu:
        agg = jnp.maximum(agg, 0.0)
    out_ref[...] = agg + x0_ref[...]


def _stats_body(h_ref, out_ref):
    h = h_ref[...]
    m = jnp.mean(h, axis=0, keepdims=True)
    v = jnp.mean((h - m) ** 2, axis=0, keepdims=True)
    out_ref[...] = jnp.concatenate([m, v], axis=0)


def _bnmm_body(bn, has_next, *refs):
    if has_next:
        if bn:
            (h_ref, st_ref, g_ref, be_ref, wn_ref, dinv_ref,
             hn_ref, tn_ref) = refs
        else:
            h_ref, wn_ref, dinv_ref, hn_ref, tn_ref = refs
    else:
        h_ref, st_ref, g_ref, be_ref, out_ref = refs
    h = h_ref[...]
    if bn:
        st = st_ref[...]
        m, v = st[0:1, :], st[1:2, :]
        h = (h - m) / jnp.sqrt(v + 1e-5) * g_ref[...] + be_ref[...]
    if has_next:
        hn = jnp.dot(h, wn_ref[...], preferred_element_type=jnp.float32,
                     precision=lax.Precision.HIGHEST)
        hn_ref[...] = hn
        tn_ref[...] = dinv_ref[...] * hn
    else:
        out_ref[...] = h


def kernel(x, edge_index, edge_weight, W0, b0, W1, b1, W2, b2,
           g0, be0, g1, be1, g2, be2):
    n, d = x.shape
    e = edge_weight.shape[0]
    n_pad = ((n + NS * 8 - 1) // (NS * 8)) * (NS * 8)

    src = edge_index[0]
    dst = edge_index[1]

    f32 = jnp.float32
    sds = jax.ShapeDtypeStruct

    ewb = pl.pallas_call(
        _ewb_body,
        grid=(e // 8 // _BLK,),
        in_specs=[pl.BlockSpec((_BLK, 8), lambda i: (i, 0))],
        out_specs=_row_spec(D),
        out_shape=sds((e // 8, D), f32),
    )(edge_weight.reshape(e // 8, 8)).reshape(e * LANES)

    h0 = _tc_call(_matmul_body, sds((n, d), f32), x, W0)

    seg_half = [_make_seg_kernel(n_pad, e // 2, h * (e // 2), 200)
                for h in range(2)]

    ones_t = jnp.ones((n, d), f32)
    dp0 = seg_half[0](ones_t, src, dst, ewb)
    dp1 = seg_half[1](ones_t, src, dst, ewb)
    dinv_col = _tc_call(functools.partial(_degfin_body, n),
                        sds((n, 1), f32), dp0, dp1)

    dinv_b, t = _tc_call(_dinv_body, [sds((n, d), f32), sds((n, d), f32)],
                         dinv_col, h0)

    b_ = [b0.reshape(1, d), b1.reshape(1, d), b2.reshape(1, d)]
    g_ = [g0.reshape(1, d), g1.reshape(1, d), g2.reshape(1, d)]
    be_ = [be0.reshape(1, d), be1.reshape(1, d), be2.reshape(1, d)]
    wn_ = [W1, W2, None]

    grid_rows = (n // _BLK,)
    h = h0
    out = None
    for i in range(3):
        p0 = seg_half[0](t, src, dst, ewb)
        p1 = seg_half[1](t, src, dst, ewb)
        relu = i < 2
        bn = i != 1
        hpre = pl.pallas_call(
            functools.partial(_pre_body, relu),
            grid=grid_rows,
            in_specs=[_row_spec(d)] * 5 + [_full_spec((1, d))],
            out_specs=_row_spec(d),
            out_shape=sds((n, d), f32),
        )(p0, p1, h, dinv_b, x, b_[i])
        if bn:
            stats = _tc_call(_stats_body, sds((2, d), f32), hpre)
        if i < 2:
            if bn:
                ins = (hpre, stats, g_[i], be_[i], wn_[i], dinv_b)
                in_specs = [_row_spec(d), _full_spec((2, d)),
                            _full_spec((1, d)), _full_spec((1, d)),
                            _full_spec((d, d)), _row_spec(d)]
            else:
                ins = (hpre, wn_[i], dinv_b)
                in_specs = [_row_spec(d), _full_spec((d, d)), _row_spec(d)]
            h, t = pl.pallas_call(
                functools.partial(_bnmm_body, bn, True),
                grid=grid_rows,
                in_specs=in_specs,
                out_specs=[_row_spec(d)] * 2,
                out_shape=[sds((n, d), f32), sds((n, d), f32)],
            )(*ins)
        else:
            out = pl.pallas_call(
                functools.partial(_bnmm_body, bn, False),
                grid=grid_rows,
                in_specs=[_row_spec(d), _full_spec((2, d)),
                          _full_spec((1, d)), _full_spec((1, d))],
                out_specs=_row_spec(d),
                out_shape=sds((n, d), f32),
            )(hpre, stats, g_[i], be_[i])
    return out
